# baseline (speedup 1.0000x reference)
import functools

import jax
import jax.numpy as jnp
from jax import lax
from jax.experimental import pallas as pl
from jax.experimental.pallas import tpu as pltpu
from jax.experimental.pallas import tpu_sc as plsc

_D = 32
_BATCH = 16384
_HIST = 50
_NC, _NS = 2, 16
_NW = _NC * _NS
_BPW = 4
_HC = 10
_NHC = _HIST // _HC
_L = 16

_mesh = plsc.VectorSubcoreMesh(core_axis_name="c", subcore_axis_name="s")


@functools.partial(
    pl.kernel,
    mesh=_mesh,
    compiler_params=pltpu.CompilerParams(
        use_tc_tiling_on_sc=False, needs_layout_passes=False
    ),
    out_type=jax.ShapeDtypeStruct((_HIST, _D // 8, _BATCH // 128, 1024),
                                  jnp.float32),
    scratch_types=[
        pltpu.VMEM((_HIST, 128 * _BPW), jnp.int32),
        pltpu.VMEM((128, _D), jnp.float32),
        pltpu.VMEM((128, _D), jnp.float32),
        pltpu.VMEM((128 * _D,), jnp.float32),
        pltpu.VMEM((_D // 8 * _HC, 1024), jnp.float32),
        pltpu.SemaphoreType.DMA,
        pltpu.SemaphoreType.DMA,
    ],
)
def _embed_gather(idx_hbm, table_hbm, out_hbm,
                  idx_v, rows0, rows1, rows_flat, slab, sem0, sem1):
    wid = lax.axis_index("s") * _NC + lax.axis_index("c")
    col0 = wid * (128 * _BPW)

    pltpu.sync_copy(idx_hbm.at[:, pl.ds(col0, 128 * _BPW)], idx_v)

    rows_bufs = (rows0, rows1)
    sems = (sem0, sem1)
    iota = lax.broadcasted_iota(jnp.int32, (_L,), 0)

    def fire(h, kb, b):
        pltpu.async_copy(
            table_hbm.at[idx_v.at[h, pl.ds(kb * 128, 128)]],
            rows_bufs[b],
            sems[b],
        )

    def drain(h, kb, b):
        pltpu.make_async_copy(
            table_hbm.at[idx_v.at[h, pl.ds(kb * 128, 128)]],
            rows_bufs[b],
            sems[b],
        ).wait()

    iota32 = iota * _D

    def transpose_into_slab(b, hh):
        for b2 in range(128):
            for k0 in range(_D // _L):
                rows_flat[pl.ds(b2 * _D + k0 * _L, _L)] = (
                    rows_bufs[b][b2, pl.ds(k0 * _L, _L)]
                )
        for ci in range(_D // 8):
            row = ci * _HC + hh
            for cc in range(8):
                c = 8 * ci + cc
                for k in range(128 // _L):
                    vals = plsc.load_gather(
                        rows_flat, [iota32 + (_L * k * _D + c)]
                    )
                    slab[row, pl.ds(cc * 128 + _L * k, _L)] = vals

    def unit_body(t, carry):
        kb = t // _NHC
        hc = t - kb * _NHC
        h0 = hc * _HC

        fire(h0, kb, 0)

        def pair(p, c2):
            for b in range(2):
                hh = 2 * p + b
                h = h0 + hh

                @pl.when(hh + 1 < _HC)
                def _():
                    fire(h + 1, kb, 1 - b)

                drain(h, kb, b)
                transpose_into_slab(b, hh)
            return c2

        lax.fori_loop(0, _HC // 2, pair, 0)

        for ci in range(_D // 8):
            pltpu.sync_copy(
                slab.at[pl.ds(ci * _HC, _HC)],
                out_hbm.at[pl.ds(h0, _HC), ci, wid * _BPW + kb],
            )
        return carry

    lax.fori_loop(0, _BPW * _NHC, unit_body, 0)


def kernel(item_inputs, table):
    idx_t = item_inputs.astype(jnp.int32).T
    out4 = _embed_gather(idx_t, table)
    out5 = out4.reshape(_HIST, _D // 8, _BATCH // 128, 8, 128)
    return out5.transpose(2, 4, 0, 1, 3).reshape(_BATCH, _HIST, _D)


# speedup vs baseline: 1.2659x; 1.2659x over previous
import functools

import jax
import jax.numpy as jnp
from jax import lax
from jax.experimental import pallas as pl
from jax.experimental.pallas import tpu as pltpu
from jax.experimental.pallas import tpu_sc as plsc

_D = 32
_BATCH = 16384
_HIST = 50
_HP = 56
_NC, _NS = 2, 16
_NW = _NC * _NS
_RPW = _BATCH // _NW
_R = 16
_NG = _RPW // _R

_mesh = plsc.VectorSubcoreMesh(core_axis_name="c", subcore_axis_name="s")


@functools.partial(
    pl.kernel,
    mesh=_mesh,
    compiler_params=pltpu.CompilerParams(use_tc_tiling_on_sc=False),
    out_type=jax.ShapeDtypeStruct((_BATCH, _HIST, _D), jnp.float32),
    scratch_types=[
        pltpu.VMEM((_RPW, _HP), jnp.int32),
        pltpu.VMEM((_R, _HP, _D), jnp.float32),
        pltpu.VMEM((_R, _HP, _D), jnp.float32),
        pltpu.SemaphoreType.DMA,
        pltpu.SemaphoreType.DMA,
    ],
)
def _embed_gather(idx_hbm, table_hbm, out_hbm, idx_v, rows0, rows1, sem0, sem1):
    wid = lax.axis_index("s") * _NC + lax.axis_index("c")
    row0 = wid * _RPW

    pltpu.sync_copy(idx_hbm.at[pl.ds(row0, _RPW)], idx_v)

    rows_bufs = (rows0, rows1)
    sems = (sem0, sem1)

    def fire(g, b):
        for j in range(_R):
            pltpu.async_copy(
                table_hbm.at[idx_v.at[g * _R + j]],
                rows_bufs[b].at[j],
                sems[b],
            )

    def drain(g, b):
        for j in range(_R):
            pltpu.make_async_copy(
                table_hbm.at[idx_v.at[g * _R + j]],
                rows_bufs[b].at[j],
                sems[b],
            ).wait()

    fire(0, 0)

    def pair_body(t, carry):
        for bb in range(2):
            g = 2 * t + bb

            @pl.when(g + 1 < _NG)
            def _():
                fire(g + 1, 1 - bb)

            drain(g, bb)
            pltpu.sync_copy(
                rows_bufs[bb].at[:, pl.ds(0, _HIST), :],
                out_hbm.at[pl.ds(row0 + g * _R, _R)],
            )
        return carry

    lax.fori_loop(0, _NG // 2, pair_body, 0)


def kernel(item_inputs, table):
    ii = item_inputs.astype(jnp.int32)
    idx = jnp.concatenate([ii, ii[:, : _HP - _HIST]], axis=1)
    return _embed_gather(idx, table)
